# baseline (speedup 1.0000x reference)
import functools

import jax
import jax.numpy as jnp
from jax import lax
from jax.experimental import pallas as pl
from jax.experimental.pallas import tpu as pltpu
from jax.experimental.pallas import tpu_sc as plsc

TOPK = 8
NE = 64
D = 2048
NT = 16384

NC = 2
NS = 16
L = 16
NW = NC * NS
TPW = NT // NW
TPH = TPW // 2
NGH = TPH // L

BT = 1024

_NEG = -1e30

_BATCHER8 = ((0, 1), (2, 3), (4, 5), (6, 7),
             (0, 2), (1, 3), (4, 6), (5, 7),
             (1, 2), (5, 6),
             (0, 4), (1, 5), (2, 6), (3, 7),
             (2, 4), (3, 5),
             (1, 2), (3, 4), (5, 6))
_BITONIC8 = ((0, 4), (1, 5), (2, 6), (3, 7),
             (0, 2), (1, 3), (4, 6), (5, 7),
             (0, 1), (2, 3), (4, 5), (6, 7))


def _mm_body(x_ref, w_ref, out_ref, inv_ref):
    lt = lax.dot_general(
        w_ref[...], x_ref[...], (((1,), (1,)), ((), ())),
        preferred_element_type=jnp.float32)
    out_ref[...] = lt
    inv_ref[...] = 1.0 / jnp.sum(jnp.exp(lt), axis=0, keepdims=True)


def _logits_t(x, w_g):
    return pl.pallas_call(
        _mm_body,
        grid=(NT // BT,),
        in_specs=[
            pl.BlockSpec((BT, D), lambda i: (i, 0)),
            pl.BlockSpec((NE, D), lambda i: (0, 0)),
        ],
        out_specs=[
            pl.BlockSpec((NE, BT), lambda i: (0, i)),
            pl.BlockSpec((1, BT), lambda i: (0, i)),
        ],
        out_shape=[
            jax.ShapeDtypeStruct((NE, NT), jnp.float32),
            jax.ShapeDtypeStruct((1, NT), jnp.float32),
        ],
    )(x, w_g)


def _cmpx(v, i, a, b):
    gt = v[b] > v[a]
    va = jnp.maximum(v[a], v[b])
    vb = jnp.minimum(v[a], v[b])
    ia = jnp.where(gt, i[b], i[a])
    ib = jnp.where(gt, i[a], i[b])
    v[a], v[b], i[a], i[b] = va, vb, ia, ib


def _merge8(av, ai, bv, bi):
    cv, ci = [], []
    for j in range(TOPK):
        gt = bv[7 - j] > av[j]
        cv.append(jnp.where(gt, bv[7 - j], av[j]))
        ci.append(jnp.where(gt, bi[7 - j], ai[j]))
    for a, b in _BITONIC8:
        _cmpx(cv, ci, a, b)
    return cv, ci


_MESH = plsc.VectorSubcoreMesh(core_axis_name="c", subcore_axis_name="s")


@functools.partial(
    pl.kernel,
    mesh=_MESH,
    out_type=(
        jax.ShapeDtypeStruct((TOPK, NT), jnp.float32),
        jax.ShapeDtypeStruct((TOPK, NT), jnp.int32),
    ),
    scratch_types=[
        pltpu.VMEM((NE, TPH), jnp.float32),
        pltpu.VMEM((NE, TPH), jnp.float32),
        pltpu.VMEM((1, TPW), jnp.float32),
        pltpu.VMEM((TOPK, TPW), jnp.float32),
        pltpu.VMEM((TOPK, TPW), jnp.int32),
        pltpu.SemaphoreType.DMA,
    ],
)
def _sc_topk(lt_hbm, inv_hbm, vals_hbm, idx_hbm,
             lbuf0, lbuf1, invbuf, vbuf, ibuf, sem):
    wid = lax.axis_index("s") * NC + lax.axis_index("c")
    base = wid * TPW
    cp1 = pltpu.make_async_copy(
        lt_hbm.at[:, pl.ds(base + TPH, TPH)], lbuf1, sem)
    cp1.start()
    pltpu.sync_copy(lt_hbm.at[:, pl.ds(base, TPH)], lbuf0)
    pltpu.sync_copy(inv_hbm.at[:, pl.ds(base, TPW)], invbuf)

    def make_group(lbuf, out0):
        def group(g, carry):
            tok0 = pl.multiple_of(g * L, L)
            sl = pl.ds(tok0, L)
            osl = pl.ds(tok0 + out0, L)
            leaves = []
            for b0 in range(0, NE, 8):
                bv = [lbuf[b0 + j, sl] for j in range(8)]
                bi = [jnp.full((L,), b0 + j, jnp.int32) for j in range(8)]
                for a, b in _BATCHER8:
                    _cmpx(bv, bi, a, b)
                leaves.append((bv, bi))
            while len(leaves) > 1:
                leaves = [ _merge8(*leaves[p], *leaves[p + 1])
                           for p in range(0, len(leaves), 2) ]
            vals, idxs = leaves[0]
            inv = invbuf[0, osl]
            for j in range(TOPK):
                vbuf[j, osl] = jnp.exp(vals[j]) * inv
                ibuf[j, osl] = idxs[j]
            return carry
        return group

    lax.fori_loop(0, NGH, make_group(lbuf0, 0), 0)
    cp1.wait()
    lax.fori_loop(0, NGH, make_group(lbuf1, TPH), 0)
    pltpu.sync_copy(vbuf, vals_hbm.at[:, pl.ds(base, TPW)])
    pltpu.sync_copy(ibuf, idx_hbm.at[:, pl.ds(base, TPW)])


def kernel(x, W_g):
    lt, inv = _logits_t(x, W_g)
    vals_t, idx_t = _sc_topk(lt, inv)
    return vals_t.T, idx_t.T


# speedup vs baseline: 1.0076x; 1.0076x over previous
import functools

import jax
import jax.numpy as jnp
from jax import lax
from jax.experimental import pallas as pl
from jax.experimental.pallas import tpu as pltpu
from jax.experimental.pallas import tpu_sc as plsc

TOPK = 8
NE = 64
D = 2048
NT = 16384

NC = 2
NS = 16
L = 16
NW = NC * NS
TPW = NT // NW
NG = TPW // L

BT = 2048

_NEG = -1e30

_BATCHER8 = ((0, 1), (2, 3), (4, 5), (6, 7),
             (0, 2), (1, 3), (4, 6), (5, 7),
             (1, 2), (5, 6),
             (0, 4), (1, 5), (2, 6), (3, 7),
             (2, 4), (3, 5),
             (1, 2), (3, 4), (5, 6))
_BITONIC8 = ((0, 4), (1, 5), (2, 6), (3, 7),
             (0, 2), (1, 3), (4, 6), (5, 7),
             (0, 1), (2, 3), (4, 5), (6, 7))


def _mm_body(x_ref, w_ref, out_ref, inv_ref):
    lt = lax.dot_general(
        w_ref[...], x_ref[...], (((1,), (1,)), ((), ())),
        preferred_element_type=jnp.float32)
    out_ref[...] = lt
    inv_ref[...] = 1.0 / jnp.sum(jnp.exp(lt), axis=0, keepdims=True)


def _logits_t(x, w_g):
    return pl.pallas_call(
        _mm_body,
        grid=(NT // BT,),
        in_specs=[
            pl.BlockSpec((BT, D), lambda i: (i, 0)),
            pl.BlockSpec((NE, D), lambda i: (0, 0)),
        ],
        out_specs=[
            pl.BlockSpec((NE, BT), lambda i: (0, i)),
            pl.BlockSpec((1, BT), lambda i: (0, i)),
        ],
        out_shape=[
            jax.ShapeDtypeStruct((NE, NT), jnp.float32),
            jax.ShapeDtypeStruct((1, NT), jnp.float32),
        ],
    )(x, w_g)


def _cmpx(v, i, a, b):
    gt = v[b] > v[a]
    va = jnp.maximum(v[a], v[b])
    vb = jnp.minimum(v[a], v[b])
    ia = jnp.where(gt, i[b], i[a])
    ib = jnp.where(gt, i[a], i[b])
    v[a], v[b], i[a], i[b] = va, vb, ia, ib


_MESH = plsc.VectorSubcoreMesh(core_axis_name="c", subcore_axis_name="s")


@functools.partial(
    pl.kernel,
    mesh=_MESH,
    out_type=(
        jax.ShapeDtypeStruct((TOPK, NT), jnp.float32),
        jax.ShapeDtypeStruct((TOPK, NT), jnp.int32),
    ),
    scratch_types=[
        pltpu.VMEM((NE, TPW), jnp.float32),
        pltpu.VMEM((1, TPW), jnp.float32),
        pltpu.VMEM((TOPK, TPW), jnp.float32),
        pltpu.VMEM((TOPK, TPW), jnp.int32),
    ],
)
def _sc_topk(lt_hbm, inv_hbm, vals_hbm, idx_hbm, lbuf, invbuf, vbuf, ibuf):
    wid = lax.axis_index("s") * NC + lax.axis_index("c")
    base = wid * TPW
    pltpu.sync_copy(lt_hbm.at[:, pl.ds(base, TPW)], lbuf)
    pltpu.sync_copy(inv_hbm.at[:, pl.ds(base, TPW)], invbuf)

    def group(g, carry):
        tok0 = pl.multiple_of(g * L, L)
        sl = pl.ds(tok0, L)
        vals = [jnp.full((L,), _NEG, jnp.float32) for _ in range(TOPK)]
        idxs = [jnp.zeros((L,), jnp.int32) for _ in range(TOPK)]
        for b0 in range(0, NE, 8):
            bv = [lbuf[b0 + j, sl] for j in range(8)]
            bi = [jnp.full((L,), b0 + j, jnp.int32) for j in range(8)]
            for a, b in _BATCHER8:
                _cmpx(bv, bi, a, b)
            cv, ci = [], []
            for j in range(TOPK):
                gt = bv[7 - j] > vals[j]
                cv.append(jnp.where(gt, bv[7 - j], vals[j]))
                ci.append(jnp.where(gt, bi[7 - j], idxs[j]))
            for a, b in _BITONIC8:
                _cmpx(cv, ci, a, b)
            vals, idxs = cv, ci
        inv = invbuf[0, sl]
        for j in range(TOPK):
            vbuf[j, sl] = jnp.exp(vals[j]) * inv
            ibuf[j, sl] = idxs[j]
        return carry

    lax.fori_loop(0, NG, group, 0)
    pltpu.sync_copy(vbuf, vals_hbm.at[:, pl.ds(base, TPW)])
    pltpu.sync_copy(ibuf, idx_hbm.at[:, pl.ds(base, TPW)])


def kernel(x, W_g):
    lt, inv = _logits_t(x, W_g)
    vals_t, idx_t = _sc_topk(lt, inv)
    return vals_t.T, idx_t.T


# speedup vs baseline: 1.0266x; 1.0188x over previous
import functools

import jax
import jax.numpy as jnp
from jax import lax
from jax.experimental import pallas as pl
from jax.experimental.pallas import tpu as pltpu
from jax.experimental.pallas import tpu_sc as plsc

TOPK = 8
NE = 64
D = 2048
NT = 16384

NC = 2
NS = 16
L = 16
NW = NC * NS
TPW = NT // NW
NG = TPW // L

BT = 1024

_NEG = -1e30

_BATCHER8 = ((0, 1), (2, 3), (4, 5), (6, 7),
             (0, 2), (1, 3), (4, 6), (5, 7),
             (1, 2), (5, 6),
             (0, 4), (1, 5), (2, 6), (3, 7),
             (2, 4), (3, 5),
             (1, 2), (3, 4), (5, 6))
_BITONIC8 = ((0, 4), (1, 5), (2, 6), (3, 7),
             (0, 2), (1, 3), (4, 6), (5, 7),
             (0, 1), (2, 3), (4, 5), (6, 7))


def _mm_body(x_ref, w_ref, out_ref, inv_ref):
    lt = lax.dot_general(
        w_ref[...], x_ref[...], (((1,), (1,)), ((), ())),
        preferred_element_type=jnp.float32)
    out_ref[...] = lt
    inv_ref[...] = 1.0 / jnp.sum(jnp.exp(lt), axis=0, keepdims=True)


def _logits_t(x, w_g):
    return pl.pallas_call(
        _mm_body,
        grid=(NT // BT,),
        in_specs=[
            pl.BlockSpec((BT, D), lambda i: (i, 0)),
            pl.BlockSpec((NE, D), lambda i: (0, 0)),
        ],
        out_specs=[
            pl.BlockSpec((NE, BT), lambda i: (0, i)),
            pl.BlockSpec((1, BT), lambda i: (0, i)),
        ],
        out_shape=[
            jax.ShapeDtypeStruct((NE, NT), jnp.float32),
            jax.ShapeDtypeStruct((1, NT), jnp.float32),
        ],
    )(x, w_g)


def _cmpx(v, i, a, b):
    gt = v[b] > v[a]
    va = jnp.maximum(v[a], v[b])
    vb = jnp.minimum(v[a], v[b])
    ia = jnp.where(gt, i[b], i[a])
    ib = jnp.where(gt, i[a], i[b])
    v[a], v[b], i[a], i[b] = va, vb, ia, ib


_MESH = plsc.VectorSubcoreMesh(core_axis_name="c", subcore_axis_name="s")


@functools.partial(
    pl.kernel,
    mesh=_MESH,
    out_type=(
        jax.ShapeDtypeStruct((TOPK, NT), jnp.float32),
        jax.ShapeDtypeStruct((TOPK, NT), jnp.int32),
    ),
    scratch_types=[
        pltpu.VMEM((NE, TPW), jnp.float32),
        pltpu.VMEM((1, TPW), jnp.float32),
        pltpu.VMEM((TOPK, TPW), jnp.float32),
        pltpu.VMEM((TOPK, TPW), jnp.int32),
    ],
)
def _sc_topk(lt_hbm, inv_hbm, vals_hbm, idx_hbm, lbuf, invbuf, vbuf, ibuf):
    wid = lax.axis_index("s") * NC + lax.axis_index("c")
    base = wid * TPW
    pltpu.sync_copy(lt_hbm.at[:, pl.ds(base, TPW)], lbuf)
    pltpu.sync_copy(inv_hbm.at[:, pl.ds(base, TPW)], invbuf)

    def group(g, carry):
        tok0 = pl.multiple_of(g * L, L)
        sl = pl.ds(tok0, L)
        vals = [jnp.full((L,), _NEG, jnp.float32) for _ in range(TOPK)]
        idxs = [jnp.zeros((L,), jnp.int32) for _ in range(TOPK)]
        for b0 in range(0, NE, 8):
            bv = [lbuf[b0 + j, sl] for j in range(8)]
            bi = [jnp.full((L,), b0 + j, jnp.int32) for j in range(8)]
            for a, b in _BATCHER8:
                _cmpx(bv, bi, a, b)
            cv, ci = [], []
            for j in range(TOPK):
                gt = bv[7 - j] > vals[j]
                cv.append(jnp.where(gt, bv[7 - j], vals[j]))
                ci.append(jnp.where(gt, bi[7 - j], idxs[j]))
            for a, b in _BITONIC8:
                _cmpx(cv, ci, a, b)
            vals, idxs = cv, ci
        inv = invbuf[0, sl]
        for j in range(TOPK):
            vbuf[j, sl] = jnp.exp(vals[j]) * inv
            ibuf[j, sl] = idxs[j]
        return carry

    lax.fori_loop(0, NG, group, 0)
    pltpu.sync_copy(vbuf, vals_hbm.at[:, pl.ds(base, TPW)])
    pltpu.sync_copy(ibuf, idx_hbm.at[:, pl.ds(base, TPW)])


def kernel(x, W_g):
    lt, inv = _logits_t(x, W_g)
    vals_t, idx_t = _sc_topk(lt, inv)
    return vals_t.T, idx_t.T
